# baseline (speedup 1.0000x reference)
import functools

import jax
import jax.numpy as jnp
from jax import lax
from jax.experimental import pallas as pl
from jax.experimental.pallas import tpu as pltpu
from jax.experimental.pallas import tpu_sc as plsc

N = 10000
F = 128
S = 64
E = 320000
BN_EPS = 1e-5

NC = 2
NS = 16
NW = NC * NS
K = 72
C = 140
E_PAD = NW * C * K
N_PAD = 10112
RPT = N_PAD // NS


def _sc_aggregate(h, src_r, dst_r, zeros):
    mesh = plsc.VectorSubcoreMesh(core_axis_name="c", subcore_axis_name="s")

    @functools.partial(
        pl.kernel,
        mesh=mesh,
        out_type=jax.ShapeDtypeStruct((NC, N_PAD, F), jnp.float32),
        scratch_types=[
            pltpu.VMEM((C, K), jnp.int32),
            pltpu.VMEM((C, K), jnp.int32),
            pltpu.VMEM((K, F), jnp.float32),
            pltpu.VMEM_SHARED((N_PAD, F), jnp.float32),
            pltpu.SemaphoreType.DMA,
            pltpu.SemaphoreType.DMA,
        ],
    )
    def agg(h_hbm, src_hbm, dst_hbm, z_hbm, out_hbm, src_v, dst_v, rows_v,
            acc_sh, sem, sz):
        cid = lax.axis_index("c")
        sid = lax.axis_index("s")
        wid = sid * NC + cid
        row0 = sid * RPT
        zcp = pltpu.async_copy(z_hbm.at[pl.ds(row0, RPT)],
                               acc_sh.at[pl.ds(row0, RPT)], sz)
        pltpu.sync_copy(src_hbm.at[wid], src_v)
        pltpu.sync_copy(dst_hbm.at[wid], dst_v)
        zcp.wait()
        plsc.subcore_barrier()

        def chunk(ci, carry):
            pltpu.async_copy(h_hbm.at[src_v.at[ci]], rows_v, sem).wait()
            pltpu.sync_copy(rows_v, acc_sh.at[dst_v.at[ci]], add=True)
            return carry

        lax.fori_loop(0, C, chunk, 0)
        plsc.subcore_barrier()
        pltpu.sync_copy(acc_sh.at[pl.ds(row0, RPT)],
                        out_hbm.at[cid, pl.ds(row0, RPT)])

    return agg(h, src_r, dst_r, zeros)


def _mlp_body(h_ref, p_ref, s_ref, w1_ref, b1_ref, w2_ref, b2_ref,
              wo_ref, add_ref, hout_ref, pout_ref):
    z = h_ref[...] * s_ref[...] + (p_ref[0] + p_ref[1])
    z = jnp.dot(z, w1_ref[...], preferred_element_type=jnp.float32) + b1_ref[...]
    z = jnp.maximum(z, 0.0)
    hn = jnp.dot(z, w2_ref[...], preferred_element_type=jnp.float32) + b2_ref[...]
    hn = jnp.maximum(hn, 0.0)
    hout_ref[...] = hn
    pout_ref[...] = jnp.dot(hn, wo_ref[...],
                            preferred_element_type=jnp.float32) + add_ref[...]


_BR = 1000


def _tc_layer(h, pagg, scal_row, w1f, b1f, w2, b2, wo, addin):
    rows3 = lambda i: (0, i, 0)
    rows = lambda i: (i, 0)
    full = lambda i: (0, 0)
    return pl.pallas_call(
        _mlp_body,
        grid=(N // _BR,),
        in_specs=[
            pl.BlockSpec((_BR, F), rows),
            pl.BlockSpec((2, _BR, F), rows3),
            pl.BlockSpec((1, F), full),
            pl.BlockSpec((F, F), full),
            pl.BlockSpec((1, F), full),
            pl.BlockSpec((F, F), full),
            pl.BlockSpec((1, F), full),
            pl.BlockSpec((F, S), full),
            pl.BlockSpec((_BR, S), rows),
        ],
        out_specs=[
            pl.BlockSpec((_BR, F), rows),
            pl.BlockSpec((_BR, S), rows),
        ],
        out_shape=[
            jax.ShapeDtypeStruct((N, F), jnp.float32),
            jax.ShapeDtypeStruct((N, S), jnp.float32),
        ],
    )(h, pagg, scal_row, w1f, b1f, w2, b2, wo, addin)


def kernel(x, edge_index,
           W1_0, b1_0, gamma_0, beta_0, rmean_0, rvar_0, W2_0, b2_0, eps_0,
           W1_1, b1_1, gamma_1, beta_1, rmean_1, rvar_1, W2_1, b2_1, eps_1,
           W1_2, b1_2, gamma_2, beta_2, rmean_2, rvar_2, W2_2, b2_2, eps_2,
           W_out, b_out):
    layers = [
        (W1_0, b1_0, gamma_0, beta_0, rmean_0, rvar_0, W2_0, b2_0, eps_0),
        (W1_1, b1_1, gamma_1, beta_1, rmean_1, rvar_1, W2_1, b2_1, eps_1),
        (W1_2, b1_2, gamma_2, beta_2, rmean_2, rvar_2, W2_2, b2_2, eps_2),
    ]
    pad = E_PAD - E
    src_r = jnp.concatenate(
        [edge_index[0], jnp.zeros((pad,), jnp.int32)]).reshape(NW, C, K)
    dst_r = jnp.concatenate(
        [edge_index[1],
         N + (jnp.arange(pad, dtype=jnp.int32) % (N_PAD - N))]).reshape(NW, C, K)
    zeros = jnp.zeros((N_PAD, F), jnp.float32)

    h = x
    pout = jnp.broadcast_to(b_out[None, :], (N, S))
    for l, (W1, b1, gamma, beta, rmean, rvar, W2, b2, eps) in enumerate(layers):
        s = gamma * lax.rsqrt(rvar + BN_EPS)
        w1f = W1 * s[None, :]
        b1f = ((b1 - rmean) * s + beta)[None, :]
        scal_row = (1.0 + eps) * jnp.ones((1, F), jnp.float32)
        wo = lax.dynamic_slice_in_dim(W_out, l * F, F, axis=0)

        pagg = _sc_aggregate(h, src_r, dst_r, zeros)
        h, pout = _tc_layer(h, pagg, scal_row,
                            w1f, b1f, W2, b2[None, :], wo, pout)
    return pout


# speedup vs baseline: 1.6493x; 1.6493x over previous
import functools

import jax
import jax.numpy as jnp
from jax import lax
from jax.experimental import pallas as pl
from jax.experimental.pallas import tpu as pltpu
from jax.experimental.pallas import tpu_sc as plsc

N = 10000
F = 128
S = 64
E = 320000
BN_EPS = 1e-5

NC = 2
NS = 16
NW = NC * NS
K = 128
C = 79
E_PAD = NW * C * K
N_PAD = 10112
RPT = N_PAD // NS


def _sc_aggregate(h, src_r, dst_r, zeros):
    mesh = plsc.VectorSubcoreMesh(core_axis_name="c", subcore_axis_name="s")

    @functools.partial(
        pl.kernel,
        mesh=mesh,
        out_type=jax.ShapeDtypeStruct((NC, N_PAD, F), jnp.float32),
        scratch_types=[
            pltpu.VMEM((C, K), jnp.int32),
            pltpu.VMEM((C, K), jnp.int32),
            pltpu.VMEM((K, F), jnp.float32),
            pltpu.VMEM_SHARED((N_PAD, F), jnp.float32),
            pltpu.SemaphoreType.DMA,
            pltpu.SemaphoreType.DMA,
        ],
    )
    def agg(h_hbm, src_hbm, dst_hbm, z_hbm, out_hbm, src_v, dst_v, rows_v,
            acc_sh, sem, sz):
        cid = lax.axis_index("c")
        sid = lax.axis_index("s")
        wid = sid * NC + cid
        row0 = sid * RPT
        zcp = pltpu.async_copy(z_hbm.at[pl.ds(row0, RPT)],
                               acc_sh.at[pl.ds(row0, RPT)], sz)
        pltpu.sync_copy(src_hbm.at[wid], src_v)
        pltpu.sync_copy(dst_hbm.at[wid], dst_v)
        zcp.wait()
        plsc.subcore_barrier()

        def chunk(ci, carry):
            pltpu.async_copy(h_hbm.at[src_v.at[ci]], rows_v, sem).wait()
            pltpu.sync_copy(rows_v, acc_sh.at[dst_v.at[ci]], add=True)
            return carry

        lax.fori_loop(0, C, chunk, 0)
        plsc.subcore_barrier()
        pltpu.sync_copy(acc_sh.at[pl.ds(row0, RPT)],
                        out_hbm.at[cid, pl.ds(row0, RPT)])

    return agg(h, src_r, dst_r, zeros)


def _mlp_body(h_ref, p_ref, s_ref, w1_ref, b1_ref, w2_ref, b2_ref,
              wo_ref, add_ref, hout_ref, pout_ref):
    z = h_ref[...] * s_ref[...] + (p_ref[0] + p_ref[1])
    z = jnp.dot(z, w1_ref[...], preferred_element_type=jnp.float32) + b1_ref[...]
    z = jnp.maximum(z, 0.0)
    hn = jnp.dot(z, w2_ref[...], preferred_element_type=jnp.float32) + b2_ref[...]
    hn = jnp.maximum(hn, 0.0)
    hout_ref[...] = hn
    pout_ref[...] = jnp.dot(hn, wo_ref[...],
                            preferred_element_type=jnp.float32) + add_ref[...]


_BR = 1000


def _tc_layer(h, pagg, scal_col, w1f, b1f, w2, b2, wo, addin):
    rows3 = lambda i: (0, i, 0)
    rows = lambda i: (i, 0)
    full = lambda i: (0, 0)
    return pl.pallas_call(
        _mlp_body,
        grid=(N // _BR,),
        in_specs=[
            pl.BlockSpec((_BR, F), rows),
            pl.BlockSpec((2, _BR, F), rows3),
            pl.BlockSpec((_BR, 1), rows),
            pl.BlockSpec((F, F), full),
            pl.BlockSpec((1, F), full),
            pl.BlockSpec((F, F), full),
            pl.BlockSpec((1, F), full),
            pl.BlockSpec((F, S), full),
            pl.BlockSpec((_BR, S), rows),
        ],
        out_specs=[
            pl.BlockSpec((_BR, F), rows),
            pl.BlockSpec((_BR, S), rows),
        ],
        out_shape=[
            jax.ShapeDtypeStruct((N, F), jnp.float32),
            jax.ShapeDtypeStruct((N, S), jnp.float32),
        ],
    )(h, pagg, scal_col, w1f, b1f, w2, b2, wo, addin)


def kernel(x, edge_index,
           W1_0, b1_0, gamma_0, beta_0, rmean_0, rvar_0, W2_0, b2_0, eps_0,
           W1_1, b1_1, gamma_1, beta_1, rmean_1, rvar_1, W2_1, b2_1, eps_1,
           W1_2, b1_2, gamma_2, beta_2, rmean_2, rvar_2, W2_2, b2_2, eps_2,
           W_out, b_out):
    layers = [
        (W1_0, b1_0, gamma_0, beta_0, rmean_0, rvar_0, W2_0, b2_0, eps_0),
        (W1_1, b1_1, gamma_1, beta_1, rmean_1, rvar_1, W2_1, b2_1, eps_1),
        (W1_2, b1_2, gamma_2, beta_2, rmean_2, rvar_2, W2_2, b2_2, eps_2),
    ]
    pad = E_PAD - E
    pad_rows = jnp.arange(pad, dtype=jnp.int32) % N
    src_r = jnp.concatenate([edge_index[0], pad_rows]).reshape(NW, C, K)
    dst_r = jnp.concatenate([edge_index[1], pad_rows]).reshape(NW, C, K)
    pad_cnt = jnp.zeros((N, 1), jnp.float32).at[:pad, 0].set(1.0)
    zeros = jnp.zeros((N_PAD, F), jnp.float32)

    h = x
    pout = jnp.broadcast_to(b_out[None, :], (N, S))
    for l, (W1, b1, gamma, beta, rmean, rvar, W2, b2, eps) in enumerate(layers):
        s = gamma * lax.rsqrt(rvar + BN_EPS)
        w1f = W1 * s[None, :]
        b1f = ((b1 - rmean) * s + beta)[None, :]
        scal_col = (1.0 + eps) - pad_cnt
        wo = lax.dynamic_slice_in_dim(W_out, l * F, F, axis=0)

        pagg = _sc_aggregate(h, src_r, dst_r, zeros)
        h, pout = _tc_layer(h, pagg, scal_col,
                            w1f, b1f, W2, b2[None, :], wo, pout)
    return pout


# speedup vs baseline: 2.0962x; 1.2710x over previous
import functools

import jax
import jax.numpy as jnp
from jax import lax
from jax.experimental import pallas as pl
from jax.experimental.pallas import tpu as pltpu
from jax.experimental.pallas import tpu_sc as plsc

N = 10000
F = 128
S = 64
E = 320000
BN_EPS = 1e-5

NC = 2
NS = 16
NW = NC * NS
K = 128
C = 80
E_PAD = NW * C * K
R = 2
RI = 4
U = 4
G = C // U
N_PAD = 10112
RPT = N_PAD // NS


def _sc_aggregate(h, sd, zeros):
    mesh = plsc.VectorSubcoreMesh(core_axis_name="c", subcore_axis_name="s")

    @functools.partial(
        pl.kernel,
        mesh=mesh,
        out_type=jax.ShapeDtypeStruct((NC, N_PAD, F), jnp.float32),
        scratch_types=[
            pltpu.VMEM((RI, 2, K), jnp.int32),
            pltpu.VMEM((R, K, F), jnp.float32),
            pltpu.VMEM_SHARED((N_PAD, F), jnp.float32),
            [pltpu.SemaphoreType.DMA] * RI,
            [pltpu.SemaphoreType.DMA] * R,
            [pltpu.SemaphoreType.DMA] * R,
            pltpu.SemaphoreType.DMA,
        ],
    )
    def agg(h_hbm, sd_hbm, z_hbm, out_hbm, sd_v, ring, acc_sh, si, sg, ss, sz):
        cid = lax.axis_index("c")
        sid = lax.axis_index("s")
        wid = sid * NC + cid
        row0 = sid * RPT
        zcp = pltpu.async_copy(z_hbm.at[pl.ds(row0, RPT)],
                               acc_sh.at[pl.ds(row0, RPT)], sz)
        for j in range(RI):
            pltpu.async_copy(sd_hbm.at[wid, j], sd_v.at[j], si[j])
        for b in range(R):
            pltpu.make_async_copy(sd_hbm.at[wid, b], sd_v.at[b], si[b]).wait()
            pltpu.async_copy(h_hbm.at[sd_v.at[b, 0]], ring.at[b], sg[b])
        zcp.wait()
        plsc.subcore_barrier()

        def grp(g, carry):
            for u in range(U):
                c = g * U + u
                b = u % R
                ib = u % RI
                pb = (u - 1) % R
                pib = (u - 1) % RI
                pltpu.make_async_copy(h_hbm.at[sd_v.at[ib, 0]], ring.at[b],
                                      sg[b]).wait()
                pltpu.async_copy(ring.at[b], acc_sh.at[sd_v.at[ib, 1]], ss[b],
                                 add=True)

                @pl.when(c >= 1)
                def _():
                    pltpu.make_async_copy(ring.at[pb],
                                          acc_sh.at[sd_v.at[pib, 1]],
                                          ss[pb]).wait()

                    @pl.when(c + 1 < C)
                    def _():
                        pltpu.make_async_copy(sd_hbm.at[wid, c + 1],
                                              sd_v.at[(u + 1) % RI],
                                              si[(u + 1) % RI]).wait()
                        pltpu.async_copy(h_hbm.at[sd_v.at[(u + 1) % RI, 0]],
                                         ring.at[pb], sg[pb])

                    @pl.when(c + 3 < C)
                    def _():
                        pltpu.async_copy(sd_hbm.at[wid, c + 3],
                                         sd_v.at[(u + 3) % RI],
                                         si[(u + 3) % RI])
            return carry

        lax.fori_loop(0, G, grp, 0)
        pltpu.make_async_copy(ring.at[(C - 1) % R],
                              acc_sh.at[sd_v.at[(C - 1) % RI, 1]],
                              ss[(C - 1) % R]).wait()
        plsc.subcore_barrier()
        pltpu.sync_copy(acc_sh.at[pl.ds(row0, RPT)],
                        out_hbm.at[cid, pl.ds(row0, RPT)])

    return agg(h, sd, zeros)


def _mlp_body(h_ref, p_ref, s_ref, w1_ref, b1_ref, w2_ref, b2_ref,
              wo_ref, add_ref, hout_ref, pout_ref):
    z = h_ref[...] * s_ref[...] + (p_ref[0] + p_ref[1])
    z = jnp.dot(z, w1_ref[...], preferred_element_type=jnp.float32) + b1_ref[...]
    z = jnp.maximum(z, 0.0)
    hn = jnp.dot(z, w2_ref[...], preferred_element_type=jnp.float32) + b2_ref[...]
    hn = jnp.maximum(hn, 0.0)
    hout_ref[...] = hn
    pout_ref[...] = jnp.dot(hn, wo_ref[...],
                            preferred_element_type=jnp.float32) + add_ref[...]


_BR = 1000


def _tc_layer(h, pagg, scal_col, w1f, b1f, w2, b2, wo, addin):
    rows3 = lambda i: (0, i, 0)
    rows = lambda i: (i, 0)
    full = lambda i: (0, 0)
    return pl.pallas_call(
        _mlp_body,
        grid=(N // _BR,),
        in_specs=[
            pl.BlockSpec((_BR, F), rows),
            pl.BlockSpec((2, _BR, F), rows3),
            pl.BlockSpec((_BR, 1), rows),
            pl.BlockSpec((F, F), full),
            pl.BlockSpec((1, F), full),
            pl.BlockSpec((F, F), full),
            pl.BlockSpec((1, F), full),
            pl.BlockSpec((F, S), full),
            pl.BlockSpec((_BR, S), rows),
        ],
        out_specs=[
            pl.BlockSpec((_BR, F), rows),
            pl.BlockSpec((_BR, S), rows),
        ],
        out_shape=[
            jax.ShapeDtypeStruct((N, F), jnp.float32),
            jax.ShapeDtypeStruct((N, S), jnp.float32),
        ],
    )(h, pagg, scal_col, w1f, b1f, w2, b2, wo, addin)


def kernel(x, edge_index,
           W1_0, b1_0, gamma_0, beta_0, rmean_0, rvar_0, W2_0, b2_0, eps_0,
           W1_1, b1_1, gamma_1, beta_1, rmean_1, rvar_1, W2_1, b2_1, eps_1,
           W1_2, b1_2, gamma_2, beta_2, rmean_2, rvar_2, W2_2, b2_2, eps_2,
           W_out, b_out):
    layers = [
        (W1_0, b1_0, gamma_0, beta_0, rmean_0, rvar_0, W2_0, b2_0, eps_0),
        (W1_1, b1_1, gamma_1, beta_1, rmean_1, rvar_1, W2_1, b2_1, eps_1),
        (W1_2, b1_2, gamma_2, beta_2, rmean_2, rvar_2, W2_2, b2_2, eps_2),
    ]
    pad = E_PAD - E
    pad_rows = jnp.arange(pad, dtype=jnp.int32) % N
    src_r = jnp.concatenate([edge_index[0], pad_rows]).reshape(NW, C, K)
    dst_r = jnp.concatenate([edge_index[1], pad_rows]).reshape(NW, C, K)
    sd = jnp.stack([src_r, dst_r], axis=2)
    pad_cnt = jnp.zeros((N, 1), jnp.float32).at[pad_rows, 0].add(1.0)
    zeros = jnp.zeros((N_PAD, F), jnp.float32)

    h = x
    pout = jnp.broadcast_to(b_out[None, :], (N, S))
    for l, (W1, b1, gamma, beta, rmean, rvar, W2, b2, eps) in enumerate(layers):
        s = gamma * lax.rsqrt(rvar + BN_EPS)
        w1f = W1 * s[None, :]
        b1f = ((b1 - rmean) * s + beta)[None, :]
        scal_col = (1.0 + eps) - pad_cnt
        wo = lax.dynamic_slice_in_dim(W_out, l * F, F, axis=0)

        pagg = _sc_aggregate(h, sd, zeros)
        h, pout = _tc_layer(h, pagg, scal_col,
                            w1f, b1f, W2, b2[None, :], wo, pout)
    return pout
